# baseline (speedup 1.0000x reference)
import jax
import jax.numpy as jnp
from jax import lax
from jax.experimental import pallas as pl
from jax.experimental.pallas import tpu as pltpu
from jax.experimental.pallas import tpu_sc as plsc

_B = 1024
_V = 100000
_L = 16
_NS = 16
_PER_TILE = _B // _NS
_CHUNKS = _PER_TILE // _L


def _loss_body(inT_hbm, tgt_hbm, out_hbm, tgt_v, val_v, all_v, red_v, shared, sem):
    sid = lax.axis_index("s")
    base = sid * _PER_TILE

    pltpu.sync_copy(tgt_hbm.at[pl.ds(base, _PER_TILE)], tgt_v)

    col0 = pl.multiple_of((sid // 2) * 128, 128)
    pltpu.async_copy(
        inT_hbm.at[tgt_v, pl.ds(col0, 128)], val_v, sem
    ).wait()

    acc = jnp.zeros((_L,), jnp.float32)
    cbase = (sid % 2) * _PER_TILE
    for j in range(_CHUNKS):
        row = lax.iota(jnp.int32, _L) + (j * _L)
        col = lax.iota(jnp.int32, _L) + (cbase + j * _L)
        acc = acc + plsc.load_gather(val_v, [row, col])
    red_v[...] = acc
    pltpu.sync_copy(red_v, shared.at[pl.ds(sid * _L, _L)])
    plsc.subcore_barrier()

    @pl.when(sid == 0)
    def _():
        pltpu.sync_copy(shared, all_v)
        tot = all_v[pl.ds(0, _L)]
        for i in range(1, _NS):
            tot = tot + all_v[pl.ds(i * _L, _L)]
        res = 1.0 - jnp.sum(tot) * (1.0 / _B)
        red_v[...] = jnp.full((_L,), res, jnp.float32)
        pltpu.sync_copy(red_v, out_hbm)


@jax.jit
def _loss(inT, tgt):
    mesh = plsc.VectorSubcoreMesh(
        core_axis_name="c", subcore_axis_name="s", num_cores=1
    )
    return pl.kernel(
        _loss_body,
        out_type=jax.ShapeDtypeStruct((_L,), jnp.float32),
        mesh=mesh,
        scratch_types=[
            pltpu.VMEM((_PER_TILE,), jnp.int32),
            pltpu.VMEM((_PER_TILE, 128), jnp.float32),
            pltpu.VMEM((_NS * _L,), jnp.float32),
            pltpu.VMEM((_L,), jnp.float32),
            pltpu.VMEM_SHARED((_NS * _L,), jnp.float32),
            pltpu.SemaphoreType.DMA,
        ],
        compiler_params=pltpu.CompilerParams(needs_layout_passes=False),
    )(inT, tgt)


def kernel(input_, target):
    out = _loss(input_.T, target.astype(jnp.int32))
    return out[0]
